# Initial kernel scaffold; baseline (speedup 1.0000x reference)
import jax
import jax.numpy as jnp
from jax.experimental import pallas as pl


def kernel(x, start_pos, freqs_cis, wq_a, wq_b, wkv_a, wkv_b, wo):
    raise NotImplementedError("write your pallas kernel here")


# speedup vs baseline: 1.7073x; 1.7073x over previous
import functools

import jax
import jax.numpy as jnp
import numpy as np
from jax.experimental import pallas as pl

DIM = 768
NH = 12
QLR = 512
KVLR = 512
NOPE = 128
ROPE = 64
VH = 128
QKD = NOPE + ROPE
BL = 64
S = 4096
_MSCALE = 0.1 * float(np.log(40.0)) + 1.0
SCALE = (QKD ** -0.5) * _MSCALE * _MSCALE

R = 256
W = 128


def _rope_perm(n_pairs):
    return np.concatenate([np.arange(n_pairs) * 2, np.arange(n_pairs) * 2 + 1])


def _attn_mask(w):
    r = jax.lax.broadcasted_iota(jnp.int32, (w, w), 0)
    c = jax.lax.broadcasted_iota(jnp.int32, (w, w), 1)
    return (r // BL == c // BL) & (c <= r)


def _ssa_body(x_ref, cs_ref, wqa_ref, wqb_ref, wkva_ref, wkvb_ref, wo_ref,
              o_ref):
    xb = x_ref[...]
    f32 = jnp.float32
    h1 = jnp.dot(xb, wqa_ref[...], preferred_element_type=f32)
    q = jnp.dot(h1.astype(jnp.bfloat16), wqb_ref[...],
                preferred_element_type=f32)
    kvp = jnp.dot(xb, wkva_ref[...], preferred_element_type=f32)
    kvb = jnp.dot(kvp[:, :KVLR].astype(jnp.bfloat16), wkvb_ref[...],
                  preferred_element_type=f32)

    c = cs_ref[:, :ROPE // 2]
    s = cs_ref[:, ROPE // 2:]
    kr = kvp[:, KVLR:KVLR + 32]
    ki = kvp[:, KVLR + 32:]
    kpe = jnp.concatenate([kr * c - ki * s, kr * s + ki * c], axis=1)

    mask = _attn_mask(W)
    outs = []
    for h in range(NH):
        qh = q[:, h * QKD:(h + 1) * QKD]
        qr = qh[:, NOPE:NOPE + 32]
        qi = qh[:, NOPE + 32:]
        qf = jnp.concatenate(
            [qh[:, :NOPE], qr * c - qi * s, qr * s + qi * c], axis=1)
        kf = jnp.concatenate([kvb[:, h * 256:h * 256 + NOPE], kpe], axis=1)
        vh = kvb[:, h * 256 + NOPE:h * 256 + NOPE + VH]
        head_out = []
        for w in range(R // W):
            qw = qf[w * W:(w + 1) * W].astype(jnp.bfloat16)
            kw = kf[w * W:(w + 1) * W].astype(jnp.bfloat16)
            sc = jax.lax.dot_general(
                qw, kw, (((1,), (1,)), ((), ())),
                preferred_element_type=f32) * SCALE
            sc = jnp.where(mask, sc, -1e30)
            m = jnp.max(sc, axis=1, keepdims=True)
            e = jnp.exp(sc - m)
            a = (e / jnp.sum(e, axis=1, keepdims=True)).astype(jnp.bfloat16)
            vw = vh[w * W:(w + 1) * W].astype(jnp.bfloat16)
            head_out.append(jnp.dot(a, vw, preferred_element_type=f32))
        outs.append(jnp.concatenate(head_out, axis=0))
    ob = jnp.concatenate(outs, axis=1).astype(jnp.bfloat16)
    o_ref[...] = jnp.dot(ob, wo_ref[...], preferred_element_type=f32)


@functools.partial(jax.jit, static_argnames=())
def _ssa(x2, cs, wqa_t, wqb_t, wkva_t, wkvb_t, wo_t):
    grid = (S // R,)
    bs = lambda shape, im: pl.BlockSpec(shape, im)
    row = lambda i: (i, 0)
    full = lambda i: (0, 0)
    return pl.pallas_call(
        _ssa_body,
        grid=grid,
        in_specs=[
            bs((R, DIM), row),
            bs((R, ROPE), row),
            bs((DIM, QLR), full),
            bs((QLR, NH * QKD), full),
            bs((DIM, KVLR + ROPE), full),
            bs((KVLR, NH * (NOPE + VH)), full),
            bs((NH * VH, DIM), full),
        ],
        out_specs=bs((R, DIM), row),
        out_shape=jax.ShapeDtypeStruct((S, DIM), jnp.float32),
    )(x2, cs, wqa_t, wqb_t, wkva_t, wkvb_t, wo_t)


def kernel(x, start_pos, freqs_cis, wq_a, wq_b, wkv_a, wkv_b, wo):
    del start_pos
    b = x.shape[0]
    x2 = x.reshape(S, DIM).astype(jnp.bfloat16)

    cs = jnp.concatenate([freqs_cis[:, :, 0], freqs_cis[:, :, 1]], axis=1)

    pp = _rope_perm(ROPE // 2)
    qperm = np.concatenate(
        [np.concatenate([h * QKD + np.arange(NOPE), h * QKD + NOPE + pp])
         for h in range(NH)])
    wqb_t = wq_b[qperm].T.astype(jnp.bfloat16)
    kperm = np.concatenate([np.arange(KVLR), KVLR + pp])
    wkva_t = wkv_a[kperm].T.astype(jnp.bfloat16)

    wqa_t = wq_a.T.astype(jnp.bfloat16)
    wkvb_t = wkv_b.T.astype(jnp.bfloat16)
    wo_t = wo.T.astype(jnp.bfloat16)

    out = _ssa(x2, cs, wqa_t, wqb_t, wkva_t, wkvb_t, wo_t)
    return out.reshape(b, S, DIM)
